# baseline (speedup 1.0000x reference)
import jax
import jax.numpy as jnp
from jax.experimental import pallas as pl
from jax.experimental.pallas import tpu as pltpu


def _gcn_block_kernel(a_ref, b_ref, o_ref, b_bf):
    @pl.when(pl.program_id(0) == 0)
    def _():
        b_bf[...] = b_ref[...].astype(jnp.bfloat16)

    a = a_ref[...].astype(jnp.bfloat16)
    acc = jnp.dot(a, b_bf[...], preferred_element_type=jnp.float32)
    o_ref[...] = jnp.where(acc >= 0, acc, 0.5 * acc)


def kernel(adj, embeds):
    n, k = adj.shape
    d = embeds.shape[1]
    bm = 200
    return pl.pallas_call(
        _gcn_block_kernel,
        grid=(n // bm,),
        in_specs=[
            pl.BlockSpec((bm, k), lambda i: (i, 0)),
            pl.BlockSpec((k, d), lambda i: (0, 0)),
        ],
        out_specs=pl.BlockSpec((bm, d), lambda i: (i, 0)),
        out_shape=jax.ShapeDtypeStruct((n, d), jnp.float32),
        scratch_shapes=[pltpu.VMEM((k, d), jnp.bfloat16)],
    )(adj, embeds)


# speedup vs baseline: 1.0378x; 1.0378x over previous
import jax
import jax.numpy as jnp
from jax.experimental import pallas as pl
from jax.experimental.pallas import tpu as pltpu


import functools

_BK = 2000


def _gcn_block_kernel(nkb, a_ref, b_ref, o_ref, b_bf):
    m = pl.program_id(0)
    kb = pl.program_id(1)

    @pl.when(m == 0)
    def _():
        b_bf[pl.ds(kb * _BK, _BK), :] = b_ref[...].astype(jnp.bfloat16)

    @pl.when(kb == nkb - 1)
    def _():
        a = a_ref[...].astype(jnp.bfloat16)
        acc = jnp.dot(a, b_bf[...], preferred_element_type=jnp.float32)
        o_ref[...] = jnp.where(acc >= 0, acc, 0.5 * acc)


def kernel(adj, embeds):
    n, kdim = adj.shape
    d = embeds.shape[1]
    bm = 400
    nkb = kdim // _BK
    return pl.pallas_call(
        functools.partial(_gcn_block_kernel, nkb),
        grid=(n // bm, nkb),
        in_specs=[
            pl.BlockSpec((bm, kdim), lambda m, kb: (m, 0)),
            pl.BlockSpec((_BK, d),
                         lambda m, kb: (jnp.where(m == 0, kb, nkb - 1), 0)),
        ],
        out_specs=pl.BlockSpec((bm, d), lambda m, kb: (m, 0)),
        out_shape=jax.ShapeDtypeStruct((n, d), jnp.float32),
        scratch_shapes=[pltpu.VMEM((kdim, d), jnp.bfloat16)],
    )(adj, embeds)


# speedup vs baseline: 1.0737x; 1.0346x over previous
import jax
import jax.numpy as jnp
from jax.experimental import pallas as pl
from jax.experimental.pallas import tpu as pltpu


def _gcn_block_kernel(a_ref, b_ref, o_ref, b_bf):
    @pl.when(pl.program_id(0) == 0)
    def _():
        b_bf[...] = b_ref[...].astype(jnp.bfloat16)

    a = a_ref[...].astype(jnp.bfloat16)
    acc = jnp.dot(a, b_bf[...], preferred_element_type=jnp.float32)
    o_ref[...] = jnp.where(acc >= 0, acc, 0.5 * acc)


def kernel(adj, embeds):
    n, kdim = adj.shape
    d = embeds.shape[1]
    bm = 320
    return pl.pallas_call(
        _gcn_block_kernel,
        grid=(pl.cdiv(n, bm),),
        in_specs=[
            pl.BlockSpec((bm, kdim), lambda m: (m, 0)),
            pl.BlockSpec((kdim, d), lambda m: (0, 0)),
        ],
        out_specs=pl.BlockSpec((bm, d), lambda m: (m, 0)),
        out_shape=jax.ShapeDtypeStruct((n, d), jnp.float32),
        scratch_shapes=[pltpu.VMEM((kdim, d), jnp.bfloat16)],
    )(adj, embeds)
